# baseline (speedup 1.0000x reference)
import jax
import jax.numpy as jnp
from jax.experimental import pallas as pl

_BN = 1000
_KP = 512
_OFF_CLS = 0
_OFF_BBOX = 128
_OFF_IOU = 448


_NSPLIT = 4


def _heads_kernel(x0_ref, x1_ref, x2_ref, x3_ref, w_ref,
                  bc_ref, bb_ref, bi_ref, s_ref, d_ref, i_ref):
    dk = x0_ref.shape[1]
    y = jnp.dot(x0_ref[...].astype(jnp.bfloat16), w_ref[0:dk, :],
                preferred_element_type=jnp.float32)
    y += jnp.dot(x1_ref[...].astype(jnp.bfloat16), w_ref[dk:2 * dk, :],
                 preferred_element_type=jnp.float32)
    y += jnp.dot(x2_ref[...].astype(jnp.bfloat16), w_ref[2 * dk:3 * dk, :],
                 preferred_element_type=jnp.float32)
    y += jnp.dot(x3_ref[...].astype(jnp.bfloat16), w_ref[3 * dk:4 * dk, :],
                 preferred_element_type=jnp.float32)
    kc = s_ref.shape[1]
    kb = d_ref.shape[1]
    ki = i_ref.shape[1]
    s_ref[...] = y[:, _OFF_CLS:_OFF_CLS + kc] + bc_ref[...]
    d_ref[...] = y[:, _OFF_BBOX:_OFF_BBOX + kb] + bb_ref[...]
    i_ref[...] = y[:, _OFF_IOU:_OFF_IOU + ki] + bi_ref[...]


def kernel(x, W_cls, b_cls, W_bbox, b_bbox, W_iou, b_iou):
    if x.ndim > 2:
        x = x.reshape(x.shape[0], -1)
    n, d = x.shape
    kc = W_cls.shape[0]
    kb = W_bbox.shape[0]
    ki = W_iou.shape[0]

    w = jnp.concatenate([
        W_cls.T, jnp.zeros((d, _OFF_BBOX - kc), jnp.float32),
        W_bbox.T, W_iou.T,
        jnp.zeros((d, _KP - _OFF_IOU - ki), jnp.float32),
    ], axis=1).astype(jnp.bfloat16)
    bc = b_cls.reshape(1, kc)
    bb = b_bbox.reshape(1, kb)
    bi = b_iou.reshape(1, ki)

    grid = (n // _BN,)
    row_block = lambda i: (i, 0)
    whole = lambda i: (0, 0)
    dk = d // _NSPLIT

    def col_chunk(j):
        return pl.BlockSpec((_BN, dk), lambda i, j=j: (i, j))

    scores, deltas, iou = pl.pallas_call(
        _heads_kernel,
        grid=grid,
        in_specs=[
            col_chunk(0), col_chunk(1), col_chunk(2), col_chunk(3),
            pl.BlockSpec((d, _KP), whole),
            pl.BlockSpec((1, kc), whole),
            pl.BlockSpec((1, kb), whole),
            pl.BlockSpec((1, ki), whole),
        ],
        out_specs=[
            pl.BlockSpec((_BN, kc), row_block),
            pl.BlockSpec((_BN, kb), row_block),
            pl.BlockSpec((_BN, ki), row_block),
        ],
        out_shape=[
            jax.ShapeDtypeStruct((n, kc), jnp.float32),
            jax.ShapeDtypeStruct((n, kb), jnp.float32),
            jax.ShapeDtypeStruct((n, ki), jnp.float32),
        ],
    )(x, x, x, x, w, bc, bb, bi)
    return scores, deltas, iou


# speedup vs baseline: 1.0298x; 1.0298x over previous
import jax
import jax.numpy as jnp
from jax.experimental import pallas as pl

_BN = 2000
_NSPLIT = 5
_BR = _BN // _NSPLIT
_KP = 512
_OFF_CLS = 0
_OFF_BBOX = 128
_OFF_IOU = 448


def _heads_kernel(x0_ref, x1_ref, x2_ref, x3_ref, x4_ref, w_ref,
                  bc_ref, bb_ref, bi_ref, s_ref, d_ref, i_ref):
    kc = s_ref.shape[1]
    kb = d_ref.shape[1]
    ki = i_ref.shape[1]
    w = w_ref[...]
    for j, x_ref in enumerate((x0_ref, x1_ref, x2_ref, x3_ref, x4_ref)):
        y = jnp.dot(x_ref[...].astype(jnp.bfloat16), w,
                    preferred_element_type=jnp.float32)
        rows = pl.ds(j * _BR, _BR)
        s_ref[rows, :] = y[:, _OFF_CLS:_OFF_CLS + kc] + bc_ref[...]
        d_ref[rows, :] = y[:, _OFF_BBOX:_OFF_BBOX + kb] + bb_ref[...]
        i_ref[rows, :] = y[:, _OFF_IOU:_OFF_IOU + ki] + bi_ref[...]


def kernel(x, W_cls, b_cls, W_bbox, b_bbox, W_iou, b_iou):
    if x.ndim > 2:
        x = x.reshape(x.shape[0], -1)
    n, d = x.shape
    kc = W_cls.shape[0]
    kb = W_bbox.shape[0]
    ki = W_iou.shape[0]

    w = jnp.concatenate([
        W_cls.T, jnp.zeros((d, _OFF_BBOX - kc), jnp.float32),
        W_bbox.T, W_iou.T,
        jnp.zeros((d, _KP - _OFF_IOU - ki), jnp.float32),
    ], axis=1).astype(jnp.bfloat16)
    bc = b_cls.reshape(1, kc)
    bb = b_bbox.reshape(1, kb)
    bi = b_iou.reshape(1, ki)

    grid = (n // _BN,)
    row_block = lambda i: (i, 0)
    whole = lambda i: (0, 0)

    def row_chunk(j):
        return pl.BlockSpec((_BR, d), lambda i, j=j: (_NSPLIT * i + j, 0))

    scores, deltas, iou = pl.pallas_call(
        _heads_kernel,
        grid=grid,
        in_specs=[
            row_chunk(0), row_chunk(1), row_chunk(2), row_chunk(3),
            row_chunk(4),
            pl.BlockSpec((d, _KP), whole),
            pl.BlockSpec((1, kc), whole),
            pl.BlockSpec((1, kb), whole),
            pl.BlockSpec((1, ki), whole),
        ],
        out_specs=[
            pl.BlockSpec((_BN, kc), row_block),
            pl.BlockSpec((_BN, kb), row_block),
            pl.BlockSpec((_BN, ki), row_block),
        ],
        out_shape=[
            jax.ShapeDtypeStruct((n, kc), jnp.float32),
            jax.ShapeDtypeStruct((n, kb), jnp.float32),
            jax.ShapeDtypeStruct((n, ki), jnp.float32),
        ],
    )(x, x, x, x, x, w, bc, bb, bi)
    return scores, deltas, iou


# speedup vs baseline: 1.0316x; 1.0018x over previous
import jax
import jax.numpy as jnp
from jax.experimental import pallas as pl
from jax.experimental.pallas import tpu as pltpu

_BN = 2000
_NSPLIT = 5
_BR = _BN // _NSPLIT
_KP = 512
_OFF_CLS = 0
_OFF_BBOX = 128
_OFF_IOU = 448


def _heads_kernel(x0_ref, x1_ref, x2_ref, x3_ref, x4_ref, w_ref,
                  bc_ref, bb_ref, bi_ref, s_ref, d_ref, i_ref):
    kc = s_ref.shape[1]
    kb = d_ref.shape[1]
    ki = i_ref.shape[1]
    w = w_ref[...]
    for j, x_ref in enumerate((x0_ref, x1_ref, x2_ref, x3_ref, x4_ref)):
        y = jnp.dot(x_ref[...].astype(jnp.bfloat16), w,
                    preferred_element_type=jnp.float32)
        rows = pl.ds(j * _BR, _BR)
        s_ref[rows, :] = y[:, _OFF_CLS:_OFF_CLS + kc] + bc_ref[...]
        d_ref[rows, :] = y[:, _OFF_BBOX:_OFF_BBOX + kb] + bb_ref[...]
        i_ref[rows, :] = y[:, _OFF_IOU:_OFF_IOU + ki] + bi_ref[...]


def kernel(x, W_cls, b_cls, W_bbox, b_bbox, W_iou, b_iou):
    if x.ndim > 2:
        x = x.reshape(x.shape[0], -1)
    n, d = x.shape
    kc = W_cls.shape[0]
    kb = W_bbox.shape[0]
    ki = W_iou.shape[0]

    w = jnp.concatenate([
        W_cls.T, jnp.zeros((d, _OFF_BBOX - kc), jnp.float32),
        W_bbox.T, W_iou.T,
        jnp.zeros((d, _KP - _OFF_IOU - ki), jnp.float32),
    ], axis=1).astype(jnp.bfloat16)
    bc = b_cls.reshape(1, kc)
    bb = b_bbox.reshape(1, kb)
    bi = b_iou.reshape(1, ki)

    grid = (n // _BN,)
    row_block = lambda i: (i, 0)
    whole = lambda i: (0, 0)

    def row_chunk(j):
        return pl.BlockSpec((_BR, d), lambda i, j=j: (_NSPLIT * i + j, 0))

    scores, deltas, iou = pl.pallas_call(
        _heads_kernel,
        grid=grid,
        in_specs=[
            row_chunk(0), row_chunk(1), row_chunk(2), row_chunk(3),
            row_chunk(4),
            pl.BlockSpec((d, _KP), whole),
            pl.BlockSpec((1, kc), whole),
            pl.BlockSpec((1, kb), whole),
            pl.BlockSpec((1, ki), whole),
        ],
        out_specs=[
            pl.BlockSpec((_BN, kc), row_block),
            pl.BlockSpec((_BN, kb), row_block),
            pl.BlockSpec((_BN, ki), row_block),
        ],
        out_shape=[
            jax.ShapeDtypeStruct((n, kc), jnp.float32),
            jax.ShapeDtypeStruct((n, kb), jnp.float32),
            jax.ShapeDtypeStruct((n, ki), jnp.float32),
        ],
        compiler_params=pltpu.CompilerParams(
            dimension_semantics=("parallel",),
            vmem_limit_bytes=100 * 1024 * 1024,
        ),
    )(x, x, x, x, x, w, bc, bb, bi)
    return scores, deltas, iou


# speedup vs baseline: 1.0768x; 1.0438x over previous
import jax
import jax.numpy as jnp
from jax.experimental import pallas as pl
from jax.experimental.pallas import tpu as pltpu

_CH = 1000
_NBUF = 4
_KP = 512
_OFF_CLS = 0
_OFF_BBOX = 128
_OFF_IOU = 448


def _heads_kernel(x_ref, w_ref, bc_ref, bb_ref, bi_ref,
                  s_ref, d_ref, i_ref,
                  xbuf, sbuf, dbuf, ibuf,
                  in_sems, s_sems, d_sems, i_sems):
    n = x_ref.shape[0]
    nchunk = n // _CH
    kc = s_ref.shape[1]
    kb = d_ref.shape[1]
    ki = i_ref.shape[1]
    w = w_ref[...]
    bc = bc_ref[...]
    bb = bb_ref[...]
    bi = bi_ref[...]

    def in_copy(c, slot):
        return pltpu.make_async_copy(
            x_ref.at[pl.ds(c * _CH, _CH), :], xbuf.at[slot], in_sems.at[slot])

    for s in range(_NBUF):
        in_copy(s, s).start()

    def body(c, carry):
        slot = jax.lax.rem(c, _NBUF)
        in_copy(c, slot).wait()
        y = jnp.dot(xbuf[slot].astype(jnp.bfloat16), w,
                    preferred_element_type=jnp.float32)

        @pl.when(c >= _NBUF)
        def _():
            pltpu.make_async_copy(sbuf.at[slot], sbuf.at[slot], s_sems.at[slot]).wait()
            pltpu.make_async_copy(dbuf.at[slot], dbuf.at[slot], d_sems.at[slot]).wait()
            pltpu.make_async_copy(ibuf.at[slot], ibuf.at[slot], i_sems.at[slot]).wait()

        sbuf[slot] = y[:, _OFF_CLS:_OFF_CLS + kc] + bc
        dbuf[slot] = y[:, _OFF_BBOX:_OFF_BBOX + kb] + bb
        ibuf[slot] = y[:, _OFF_IOU:_OFF_IOU + ki] + bi

        rows = pl.ds(c * _CH, _CH)
        pltpu.make_async_copy(sbuf.at[slot], s_ref.at[rows, :], s_sems.at[slot]).start()
        pltpu.make_async_copy(dbuf.at[slot], d_ref.at[rows, :], d_sems.at[slot]).start()
        pltpu.make_async_copy(ibuf.at[slot], i_ref.at[rows, :], i_sems.at[slot]).start()

        @pl.when(c + _NBUF < nchunk)
        def _():
            in_copy(c + _NBUF, slot).start()
        return carry

    jax.lax.fori_loop(0, nchunk, body, 0)

    for s in range(_NBUF):
        pltpu.make_async_copy(sbuf.at[s], sbuf.at[s], s_sems.at[s]).wait()
        pltpu.make_async_copy(dbuf.at[s], dbuf.at[s], d_sems.at[s]).wait()
        pltpu.make_async_copy(ibuf.at[s], ibuf.at[s], i_sems.at[s]).wait()


def kernel(x, W_cls, b_cls, W_bbox, b_bbox, W_iou, b_iou):
    if x.ndim > 2:
        x = x.reshape(x.shape[0], -1)
    n, d = x.shape
    kc = W_cls.shape[0]
    kb = W_bbox.shape[0]
    ki = W_iou.shape[0]

    w = jnp.concatenate([
        W_cls.T, jnp.zeros((d, _OFF_BBOX - kc), jnp.float32),
        W_bbox.T, W_iou.T,
        jnp.zeros((d, _KP - _OFF_IOU - ki), jnp.float32),
    ], axis=1).astype(jnp.bfloat16)
    bc = b_cls.reshape(1, kc)
    bb = b_bbox.reshape(1, kb)
    bi = b_iou.reshape(1, ki)

    any_spec = pl.BlockSpec(memory_space=pl.ANY)
    vmem_spec = pl.BlockSpec(memory_space=pltpu.VMEM)

    scores, deltas, iou = pl.pallas_call(
        _heads_kernel,
        in_specs=[any_spec, vmem_spec, vmem_spec, vmem_spec, vmem_spec],
        out_specs=[any_spec, any_spec, any_spec],
        out_shape=[
            jax.ShapeDtypeStruct((n, kc), jnp.float32),
            jax.ShapeDtypeStruct((n, kb), jnp.float32),
            jax.ShapeDtypeStruct((n, ki), jnp.float32),
        ],
        scratch_shapes=[
            pltpu.VMEM((_NBUF, _CH, d), jnp.float32),
            pltpu.VMEM((_NBUF, _CH, kc), jnp.float32),
            pltpu.VMEM((_NBUF, _CH, kb), jnp.float32),
            pltpu.VMEM((_NBUF, _CH, ki), jnp.float32),
            pltpu.SemaphoreType.DMA((_NBUF,)),
            pltpu.SemaphoreType.DMA((_NBUF,)),
            pltpu.SemaphoreType.DMA((_NBUF,)),
            pltpu.SemaphoreType.DMA((_NBUF,)),
        ],
        compiler_params=pltpu.CompilerParams(
            vmem_limit_bytes=110 * 1024 * 1024,
        ),
    )(x, w, bc, bb, bi)
    return scores, deltas, iou


# speedup vs baseline: 1.0777x; 1.0009x over previous
import jax
import jax.numpy as jnp
from jax.experimental import pallas as pl
from jax.experimental.pallas import tpu as pltpu

_CH = 1000
_NBUF = 4
_KP = 512
_OFF_CLS = 0
_OFF_BBOX = 128
_OFF_IOU = 448


def _heads_kernel(x_ref, w_ref, bc_ref, bb_ref, bi_ref,
                  s_ref, d_ref, i_ref,
                  xbuf, sbuf, dbuf, ibuf,
                  in_sems, s_sems, d_sems, i_sems):
    n = x_ref.shape[0]
    nchunk = n // _CH
    kc = s_ref.shape[1]
    kb = d_ref.shape[1]
    ki = i_ref.shape[1]
    w = w_ref[...]
    bc = bc_ref[...]
    bb = bb_ref[...]
    bi = bi_ref[...]

    def in_copy(c, slot):
        return pltpu.make_async_copy(
            x_ref.at[pl.ds(c * _CH, _CH), :], xbuf.at[slot], in_sems.at[slot])

    for s in range(_NBUF):
        in_copy(s, s).start()

    def body(c, carry):
        slot = jax.lax.rem(c, _NBUF)
        in_copy(c, slot).wait()
        y = jnp.dot(xbuf[slot].astype(jnp.bfloat16), w,
                    preferred_element_type=jnp.float32)

        @pl.when(c >= _NBUF)
        def _():
            pltpu.make_async_copy(sbuf.at[slot], sbuf.at[slot], s_sems.at[slot]).wait()
            pltpu.make_async_copy(dbuf.at[slot], dbuf.at[slot], d_sems.at[slot]).wait()
            pltpu.make_async_copy(ibuf.at[slot], ibuf.at[slot], i_sems.at[slot]).wait()

        sbuf[slot] = y[:, _OFF_CLS:_OFF_CLS + kc] + bc
        dbuf[slot] = y[:, _OFF_BBOX:_OFF_BBOX + kb] + bb
        ibuf[slot] = y[:, _OFF_IOU:_OFF_IOU + ki] + bi

        rows = pl.ds(c * _CH, _CH)
        pltpu.make_async_copy(sbuf.at[slot], s_ref.at[rows, :], s_sems.at[slot]).start(priority=1)
        pltpu.make_async_copy(dbuf.at[slot], d_ref.at[rows, :], d_sems.at[slot]).start(priority=1)
        pltpu.make_async_copy(ibuf.at[slot], i_ref.at[rows, :], i_sems.at[slot]).start(priority=1)

        @pl.when(c + _NBUF < nchunk)
        def _():
            in_copy(c + _NBUF, slot).start()
        return carry

    jax.lax.fori_loop(0, nchunk, body, 0)

    for s in range(_NBUF):
        pltpu.make_async_copy(sbuf.at[s], sbuf.at[s], s_sems.at[s]).wait()
        pltpu.make_async_copy(dbuf.at[s], dbuf.at[s], d_sems.at[s]).wait()
        pltpu.make_async_copy(ibuf.at[s], ibuf.at[s], i_sems.at[s]).wait()


def kernel(x, W_cls, b_cls, W_bbox, b_bbox, W_iou, b_iou):
    if x.ndim > 2:
        x = x.reshape(x.shape[0], -1)
    n, d = x.shape
    kc = W_cls.shape[0]
    kb = W_bbox.shape[0]
    ki = W_iou.shape[0]

    w = jnp.concatenate([
        W_cls.T, jnp.zeros((d, _OFF_BBOX - kc), jnp.float32),
        W_bbox.T, W_iou.T,
        jnp.zeros((d, _KP - _OFF_IOU - ki), jnp.float32),
    ], axis=1).astype(jnp.bfloat16)
    bc = b_cls.reshape(1, kc)
    bb = b_bbox.reshape(1, kb)
    bi = b_iou.reshape(1, ki)

    any_spec = pl.BlockSpec(memory_space=pl.ANY)
    vmem_spec = pl.BlockSpec(memory_space=pltpu.VMEM)

    scores, deltas, iou = pl.pallas_call(
        _heads_kernel,
        in_specs=[any_spec, vmem_spec, vmem_spec, vmem_spec, vmem_spec],
        out_specs=[any_spec, any_spec, any_spec],
        out_shape=[
            jax.ShapeDtypeStruct((n, kc), jnp.float32),
            jax.ShapeDtypeStruct((n, kb), jnp.float32),
            jax.ShapeDtypeStruct((n, ki), jnp.float32),
        ],
        scratch_shapes=[
            pltpu.VMEM((_NBUF, _CH, d), jnp.float32),
            pltpu.VMEM((_NBUF, _CH, kc), jnp.float32),
            pltpu.VMEM((_NBUF, _CH, kb), jnp.float32),
            pltpu.VMEM((_NBUF, _CH, ki), jnp.float32),
            pltpu.SemaphoreType.DMA((_NBUF,)),
            pltpu.SemaphoreType.DMA((_NBUF,)),
            pltpu.SemaphoreType.DMA((_NBUF,)),
            pltpu.SemaphoreType.DMA((_NBUF,)),
        ],
        compiler_params=pltpu.CompilerParams(
            vmem_limit_bytes=110 * 1024 * 1024,
        ),
    )(x, w, bc, bb, bi)
    return scores, deltas, iou


# speedup vs baseline: 1.0810x; 1.0031x over previous
import jax
import jax.numpy as jnp
from jax.experimental import pallas as pl
from jax.experimental.pallas import tpu as pltpu

_CH = 1000
_NBUF = 4
_KP = 512
_OFF_CLS = 0
_OFF_BBOX = 128
_OFF_IOU = 448


def _heads_kernel(x_ref, w_ref, bc_ref, bb_ref, bi_ref,
                  s_ref, d_ref, i_ref,
                  xbuf, in_sems):
    n = x_ref.shape[0]
    nchunk = n // _CH
    kc = s_ref.shape[1]
    kb = d_ref.shape[1]
    ki = i_ref.shape[1]
    c = pl.program_id(0)

    def in_copy(k, slot):
        return pltpu.make_async_copy(
            x_ref.at[pl.ds(k * _CH, _CH), :], xbuf.at[slot], in_sems.at[slot])

    @pl.when(c == 0)
    def _():
        for s in range(_NBUF):
            in_copy(s, s).start()

    slot = jax.lax.rem(c, _NBUF)
    in_copy(c, slot).wait()
    y = jnp.dot(xbuf[slot].astype(jnp.bfloat16), w_ref[...],
                preferred_element_type=jnp.float32)
    s_ref[...] = y[:, _OFF_CLS:_OFF_CLS + kc] + bc_ref[...]
    d_ref[...] = y[:, _OFF_BBOX:_OFF_BBOX + kb] + bb_ref[...]
    i_ref[...] = y[:, _OFF_IOU:_OFF_IOU + ki] + bi_ref[...]

    @pl.when(c + _NBUF < nchunk)
    def _():
        in_copy(c + _NBUF, slot).start()


def kernel(x, W_cls, b_cls, W_bbox, b_bbox, W_iou, b_iou):
    if x.ndim > 2:
        x = x.reshape(x.shape[0], -1)
    n, d = x.shape
    kc = W_cls.shape[0]
    kb = W_bbox.shape[0]
    ki = W_iou.shape[0]

    w = jnp.concatenate([
        W_cls.T, jnp.zeros((d, _OFF_BBOX - kc), jnp.float32),
        W_bbox.T, W_iou.T,
        jnp.zeros((d, _KP - _OFF_IOU - ki), jnp.float32),
    ], axis=1).astype(jnp.bfloat16)
    bc = b_cls.reshape(1, kc)
    bb = b_bbox.reshape(1, kb)
    bi = b_iou.reshape(1, ki)

    nchunk = n // _CH
    row_block = lambda i: (i, 0)
    whole = lambda i: (0, 0)

    scores, deltas, iou = pl.pallas_call(
        _heads_kernel,
        grid=(nchunk,),
        in_specs=[
            pl.BlockSpec(memory_space=pl.ANY),
            pl.BlockSpec((d, _KP), whole),
            pl.BlockSpec((1, kc), whole),
            pl.BlockSpec((1, kb), whole),
            pl.BlockSpec((1, ki), whole),
        ],
        out_specs=[
            pl.BlockSpec((_CH, kc), row_block),
            pl.BlockSpec((_CH, kb), row_block),
            pl.BlockSpec((_CH, ki), row_block),
        ],
        out_shape=[
            jax.ShapeDtypeStruct((n, kc), jnp.float32),
            jax.ShapeDtypeStruct((n, kb), jnp.float32),
            jax.ShapeDtypeStruct((n, ki), jnp.float32),
        ],
        scratch_shapes=[
            pltpu.VMEM((_NBUF, _CH, d), jnp.float32),
            pltpu.SemaphoreType.DMA((_NBUF,)),
        ],
        compiler_params=pltpu.CompilerParams(
            vmem_limit_bytes=110 * 1024 * 1024,
        ),
    )(x, w, bc, bb, bi)
    return scores, deltas, iou


# speedup vs baseline: 1.0823x; 1.0011x over previous
import jax
import jax.numpy as jnp
from jax.experimental import pallas as pl
from jax.experimental.pallas import tpu as pltpu

_CH = 1000
_NBUF = 4
_KP = 512
_OFF_CLS = 0
_OFF_BBOX = 128
_OFF_IOU = 448


def _heads_kernel(x_ref, w_ref, bc_ref, bb_ref, bi_ref,
                  s_ref, d_ref, i_ref,
                  xbuf, in_sems):
    n = x_ref.shape[0]
    nchunk = n // _CH
    kc = s_ref.shape[1]
    kb = d_ref.shape[1]
    ki = i_ref.shape[1]
    c = pl.program_id(0)

    def in_copy(k, slot):
        return pltpu.make_async_copy(
            x_ref.at[pl.ds(k * _CH, _CH), :], xbuf.at[slot], in_sems.at[slot])

    @pl.when(c == 0)
    def _():
        for s in range(_NBUF):
            in_copy(s, s).start()

    slot = jax.lax.rem(c, _NBUF)
    in_copy(c, slot).wait()
    y = jnp.dot(xbuf[slot].astype(jnp.bfloat16), w_ref[...],
                preferred_element_type=jnp.float32)
    s_ref[...] = y[:, _OFF_CLS:_OFF_CLS + kc] + bc_ref[...]
    d_ref[...] = y[:, _OFF_BBOX:_OFF_BBOX + kb] + bb_ref[...]
    i_ref[...] = y[:, _OFF_IOU:_OFF_IOU + ki] + bi_ref[...]

    @pl.when(c + _NBUF < nchunk)
    def _():
        in_copy(c + _NBUF, slot).start()


def kernel(x, W_cls, b_cls, W_bbox, b_bbox, W_iou, b_iou):
    if x.ndim > 2:
        x = x.reshape(x.shape[0], -1)
    n, d = x.shape
    kc = W_cls.shape[0]
    kb = W_bbox.shape[0]
    ki = W_iou.shape[0]

    w = jnp.concatenate([
        W_cls.T, jnp.zeros((d, _OFF_BBOX - kc), jnp.float32),
        W_bbox.T, W_iou.T,
        jnp.zeros((d, _KP - _OFF_IOU - ki), jnp.float32),
    ], axis=1).astype(jnp.bfloat16)
    bc = b_cls.reshape(1, kc)
    bb = b_bbox.reshape(1, kb)
    bi = b_iou.reshape(1, ki)

    nchunk = n // _CH
    row_block = lambda i: (i, 0)
    whole = lambda i: (0, 0)

    scores, deltas, iou = pl.pallas_call(
        _heads_kernel,
        grid=(nchunk,),
        in_specs=[
            pl.BlockSpec(memory_space=pl.ANY),
            pl.BlockSpec((d, _KP), whole),
            pl.BlockSpec((1, kc), whole),
            pl.BlockSpec((1, kb), whole),
            pl.BlockSpec((1, ki), whole),
        ],
        out_specs=[
            pl.BlockSpec((_CH, kc), row_block),
            pl.BlockSpec((_CH, kb), row_block),
            pl.BlockSpec((_CH, ki), row_block),
        ],
        out_shape=[
            jax.ShapeDtypeStruct((n, kc), jnp.float32),
            jax.ShapeDtypeStruct((n, kb), jnp.float32),
            jax.ShapeDtypeStruct((n, ki), jnp.float32),
        ],
        scratch_shapes=[
            pltpu.VMEM((_NBUF, _CH, d), jnp.float32),
            pltpu.SemaphoreType.DMA((_NBUF,)),
        ],
        compiler_params=pltpu.CompilerParams(
            vmem_limit_bytes=110 * 1024 * 1024,
            needs_layout_passes=False,
        ),
    )(x, w, bc, bb, bi)
    return scores, deltas, iou
